# Initial kernel scaffold; baseline (speedup 1.0000x reference)
import jax
import jax.numpy as jnp
from jax.experimental import pallas as pl


def kernel(node_representation, graph_index, W, b):
    raise NotImplementedError("write your pallas kernel here")


# speedup vs baseline: 4.7860x; 4.7860x over previous
import functools

import jax
import jax.numpy as jnp
from jax import lax
from jax.experimental import pallas as pl
from jax.experimental.pallas import tpu as pltpu
from jax.experimental.pallas import tpu_sc as plsc

N = 100000
D = 128
G = 512
TC_OUT = 24

NC = 2
NS = 16
NW = NC * NS

B = 400
CH = 100
NCH = B // CH
NB = N // B
ROWS_PER_TILE = G // NS

HB = 2000
NHB = N // HB


def _sc_body(nodes_hbm, gi_hbm, sums_hbm,
             rows_v, idx_v, zrow_v, acc_sh):
  cid = lax.axis_index("c")
  sid = lax.axis_index("s")
  wid = cid * NS + sid

  def fill_zrow(i, _):
    for j in range(D // 16):
      zrow_v[i, pl.ds(j * 16, 16)] = jnp.zeros((16,), jnp.float32)
    return 0
  lax.fori_loop(0, ROWS_PER_TILE, fill_zrow, 0)

  pltpu.sync_copy(zrow_v, acc_sh.at[pl.ds(sid * ROWS_PER_TILE, ROWS_PER_TILE)])
  plsc.subcore_barrier()

  nb = (NB - wid + NW - 1) // NW

  def batch_body(k, _):
    bt = wid + k * NW
    pltpu.sync_copy(gi_hbm.at[bt], idx_v)
    pltpu.sync_copy(nodes_hbm.at[pl.ds(bt * B, B)], rows_v)
    for j in range(NCH):
      pltpu.sync_copy(rows_v.at[pl.ds(j * CH, CH)],
                      acc_sh.at[idx_v.at[j]], add=True)
    return 0

  lax.fori_loop(0, nb, batch_body, 0)
  plsc.subcore_barrier()

  out_base = cid * G + sid * ROWS_PER_TILE
  pltpu.sync_copy(acc_sh.at[pl.ds(sid * ROWS_PER_TILE, ROWS_PER_TILE)],
                  sums_hbm.at[pl.ds(out_base, ROWS_PER_TILE)])


@jax.jit
def _segment_accumulate(nodes, gi_r):
  mesh = plsc.VectorSubcoreMesh(core_axis_name="c", subcore_axis_name="s",
                                num_cores=NC, num_subcores=NS)
  return pl.kernel(
      _sc_body,
      out_type=jax.ShapeDtypeStruct((NC * G, D), jnp.float32),
      mesh=mesh,
      scratch_types=[
          pltpu.VMEM((B, D), jnp.float32),
          pltpu.VMEM((NCH, CH), jnp.int32),
          pltpu.VMEM((ROWS_PER_TILE, D), jnp.float32),
          pltpu.VMEM_SHARED((G, D), jnp.float32),
      ],
  )(nodes, gi_r)


def _head_body(sums_ref, gi_ref, w_ref, b_ref, out_ref, cnt_ref):
  i = pl.program_id(0)
  gi_blk = gi_ref[0, 0, :]
  eq = (gi_blk[:, None] == lax.broadcasted_iota(jnp.int32, (HB, G), 1))
  bc = jnp.sum(eq.astype(jnp.float32), axis=0)

  @pl.when(i == 0)
  def _():
    cnt_ref[0, :] = bc

  @pl.when(i > 0)
  def _():
    cnt_ref[0, :] = cnt_ref[0, :] + bc

  @pl.when(i == NHB - 1)
  def _():
    s = sums_ref[0:G, :] + sums_ref[G:2 * G, :]
    c = jnp.reshape(cnt_ref[0, :], (G, 1))
    pooled = s / jnp.maximum(c, 1.0)
    out_ref[...] = (
        jnp.dot(pooled, w_ref[...], preferred_element_type=jnp.float32)
        + b_ref[...])


@jax.jit
def _head(sums, gi3, W, b):
  return pl.pallas_call(
      _head_body,
      grid=(NHB,),
      in_specs=[
          pl.BlockSpec((NC * G, D), lambda i: (0, 0)),
          pl.BlockSpec((1, 1, HB), lambda i: (i, 0, 0)),
          pl.BlockSpec((D, TC_OUT), lambda i: (0, 0)),
          pl.BlockSpec((1, TC_OUT), lambda i: (0, 0)),
      ],
      out_specs=pl.BlockSpec((G, TC_OUT), lambda i: (0, 0)),
      scratch_shapes=[pltpu.VMEM((1, G), jnp.float32)],
      out_shape=jax.ShapeDtypeStruct((G, TC_OUT), jnp.float32),
  )(sums, gi3, W, b.reshape(1, TC_OUT))


def kernel(node_representation, graph_index, W, b):
  gi = graph_index.astype(jnp.int32)
  gi_r = gi.reshape(NB, NCH, CH)
  gi3 = gi.reshape(NHB, 1, HB)
  sums = _segment_accumulate(node_representation, gi_r)
  out = _head(sums, gi3, W, b)
  return out.reshape(-1, TC_OUT // 2, 2)


# speedup vs baseline: 6.8978x; 1.4413x over previous
import functools

import jax
import jax.numpy as jnp
from jax import lax
from jax.experimental import pallas as pl
from jax.experimental.pallas import tpu as pltpu
from jax.experimental.pallas import tpu_sc as plsc

N = 100000
D = 128
G = 512
TC_OUT = 24

NC = 2
NS = 16
NW = NC * NS

B = 400
CH = 100
NCH = B // CH
NB = N // B
ROWS_PER_TILE = G // NS

HB = 2000
NHB = N // HB


def _sc_body(nodes_hbm, gi_hbm, sums_hbm,
             rows_v, idx_v, zrow_v, acc_sh):
  cid = lax.axis_index("c")
  sid = lax.axis_index("s")
  wid = cid * NS + sid

  def fill_zrow(i, _):
    for j in range(D // 16):
      zrow_v[i, pl.ds(j * 16, 16)] = jnp.zeros((16,), jnp.float32)
    return 0
  lax.fori_loop(0, ROWS_PER_TILE, fill_zrow, 0)

  pltpu.sync_copy(zrow_v, acc_sh.at[pl.ds(sid * ROWS_PER_TILE, ROWS_PER_TILE)])
  plsc.subcore_barrier()

  nb = (NB - wid + NW - 1) // NW

  def batch_body(k, _):
    bt = wid + k * NW
    pltpu.sync_copy(gi_hbm.at[bt], idx_v)
    pltpu.sync_copy(nodes_hbm.at[pl.ds(bt * B, B)], rows_v)
    for j in range(NCH):
      pltpu.sync_copy(rows_v.at[pl.ds(j * CH, CH)],
                      acc_sh.at[idx_v.at[j]], add=True)
    return 0

  lax.fori_loop(0, nb, batch_body, 0)
  plsc.subcore_barrier()

  out_base = cid * G + sid * ROWS_PER_TILE
  pltpu.sync_copy(acc_sh.at[pl.ds(sid * ROWS_PER_TILE, ROWS_PER_TILE)],
                  sums_hbm.at[pl.ds(out_base, ROWS_PER_TILE)])


@jax.jit
def _segment_accumulate(nodes, gi_r):
  mesh = plsc.VectorSubcoreMesh(core_axis_name="c", subcore_axis_name="s",
                                num_cores=NC, num_subcores=NS)
  return pl.kernel(
      _sc_body,
      out_type=jax.ShapeDtypeStruct((NC * G, D), jnp.float32),
      mesh=mesh,
      scratch_types=[
          pltpu.VMEM((B, D), jnp.float32),
          pltpu.VMEM((NCH, CH), jnp.int32),
          pltpu.VMEM((ROWS_PER_TILE, D), jnp.float32),
          pltpu.VMEM_SHARED((G, D), jnp.float32),
      ],
  )(nodes, gi_r)


def _hist_body(gi_ref, cnt_ref):
  i = pl.program_id(0)
  gi_blk = gi_ref[0, 0, :]
  eq = (gi_blk[:, None] == lax.broadcasted_iota(jnp.int32, (HB, G), 1))
  bc = jnp.sum(eq.astype(jnp.float32), axis=0)

  @pl.when(i == 0)
  def _():
    cnt_ref[0, :] = bc

  @pl.when(i > 0)
  def _():
    cnt_ref[0, :] = cnt_ref[0, :] + bc


@jax.jit
def _hist(gi3):
  return pl.pallas_call(
      _hist_body,
      grid=(NHB,),
      in_specs=[pl.BlockSpec((1, 1, HB), lambda i: (i, 0, 0))],
      out_specs=pl.BlockSpec((1, G), lambda i: (0, 0)),
      out_shape=jax.ShapeDtypeStruct((1, G), jnp.float32),
  )(gi3)


def _head_body(sums_ref, cnt_ref, w_ref, b_ref, out_ref):
  s = sums_ref[0:G, :] + sums_ref[G:2 * G, :]
  c = jnp.reshape(cnt_ref[0, :], (G, 1))
  pooled = s / jnp.maximum(c, 1.0)
  out_ref[...] = (
      jnp.dot(pooled, w_ref[...], preferred_element_type=jnp.float32)
      + b_ref[...])


@jax.jit
def _head(sums, cnts, W, b):
  return pl.pallas_call(
      _head_body,
      out_shape=jax.ShapeDtypeStruct((G, TC_OUT), jnp.float32),
  )(sums, cnts, W, b.reshape(1, TC_OUT))


def kernel(node_representation, graph_index, W, b):
  gi = graph_index.astype(jnp.int32)
  gi_r = gi.reshape(NB, NCH, CH)
  gi3 = gi.reshape(NHB, 1, HB)
  sums = _segment_accumulate(node_representation, gi_r)
  cnts = _hist(gi3)
  out = _head(sums, cnts, W, b)
  return out.reshape(-1, TC_OUT // 2, 2)


# speedup vs baseline: 8.1905x; 1.1874x over previous
import functools

import jax
import jax.numpy as jnp
from jax import lax
from jax.experimental import pallas as pl
from jax.experimental.pallas import tpu as pltpu
from jax.experimental.pallas import tpu_sc as plsc

N = 100000
D = 128
G = 512
TC_OUT = 24

NC = 2
NS = 16
NW = NC * NS

B = 400
CH = 100
NCH = B // CH
NB = N // B
ROWS_PER_TILE = G // NS

HB = 2000
NHB = N // HB


def _sc_body(nodes_hbm, gi_hbm, sums_hbm,
             rows_v0, rows_v1, idx_v0, idx_v1, zrow_v, acc_sh,
             lsem0, lsem1, ssem0, ssem1):
  cid = lax.axis_index("c")
  sid = lax.axis_index("s")
  wid = cid * NS + sid

  def fill_zrow(i, _):
    for j in range(D // 16):
      zrow_v[i, pl.ds(j * 16, 16)] = jnp.zeros((16,), jnp.float32)
    return 0
  lax.fori_loop(0, ROWS_PER_TILE, fill_zrow, 0)

  pltpu.sync_copy(zrow_v, acc_sh.at[pl.ds(sid * ROWS_PER_TILE, ROWS_PER_TILE)])
  plsc.subcore_barrier()

  nb = (NB - wid + NW - 1) // NW

  def issue_load(k, rows_v, idx_v, lsem):
    bt = wid + k * NW
    pltpu.async_copy(gi_hbm.at[bt], idx_v, lsem)
    pltpu.async_copy(nodes_hbm.at[pl.ds(bt * B, B)], rows_v, lsem)

  def half(k, rows_v, idx_v, lsem, ssem):
    pltpu.make_async_copy(gi_hbm.at[0], idx_v, lsem).wait()
    pltpu.make_async_copy(nodes_hbm.at[pl.ds(0, B)], rows_v, lsem).wait()
    descs = []
    for j in range(NCH):
      descs.append(pltpu.async_copy(rows_v.at[pl.ds(j * CH, CH)],
                                    acc_sh.at[idx_v.at[j]], ssem, add=True))
    for d_ in descs:
      d_.wait()

    @pl.when(k + 2 < nb)
    def _():
      issue_load(k + 2, rows_v, idx_v, lsem)

  issue_load(0, rows_v0, idx_v0, lsem0)
  issue_load(1, rows_v1, idx_v1, lsem1)

  def pair_body(k2, _):
    k = 2 * k2

    @pl.when(k < nb)
    def _():
      half(k, rows_v0, idx_v0, lsem0, ssem0)

    @pl.when(k + 1 < nb)
    def _():
      half(k + 1, rows_v1, idx_v1, lsem1, ssem1)
    return 0

  lax.fori_loop(0, (nb + 1) // 2, pair_body, 0)
  plsc.subcore_barrier()

  out_base = cid * G + sid * ROWS_PER_TILE
  pltpu.sync_copy(acc_sh.at[pl.ds(sid * ROWS_PER_TILE, ROWS_PER_TILE)],
                  sums_hbm.at[pl.ds(out_base, ROWS_PER_TILE)])


@jax.jit
def _segment_accumulate(nodes, gi_r):
  mesh = plsc.VectorSubcoreMesh(core_axis_name="c", subcore_axis_name="s",
                                num_cores=NC, num_subcores=NS)
  return pl.kernel(
      _sc_body,
      out_type=jax.ShapeDtypeStruct((NC * G, D), jnp.float32),
      mesh=mesh,
      scratch_types=[
          pltpu.VMEM((B, D), jnp.float32),
          pltpu.VMEM((B, D), jnp.float32),
          pltpu.VMEM((NCH, CH), jnp.int32),
          pltpu.VMEM((NCH, CH), jnp.int32),
          pltpu.VMEM((ROWS_PER_TILE, D), jnp.float32),
          pltpu.VMEM_SHARED((G, D), jnp.float32),
          pltpu.SemaphoreType.DMA,
          pltpu.SemaphoreType.DMA,
          pltpu.SemaphoreType.DMA,
          pltpu.SemaphoreType.DMA,
      ],
  )(nodes, gi_r)


def _hist_body(gi_ref, cnt_ref):
  i = pl.program_id(0)
  gi_blk = gi_ref[0, 0, :]
  eq = (gi_blk[:, None] == lax.broadcasted_iota(jnp.int32, (HB, G), 1))
  bc = jnp.sum(eq.astype(jnp.float32), axis=0)

  @pl.when(i == 0)
  def _():
    cnt_ref[0, :] = bc

  @pl.when(i > 0)
  def _():
    cnt_ref[0, :] = cnt_ref[0, :] + bc


@jax.jit
def _hist(gi3):
  return pl.pallas_call(
      _hist_body,
      grid=(NHB,),
      in_specs=[pl.BlockSpec((1, 1, HB), lambda i: (i, 0, 0))],
      out_specs=pl.BlockSpec((1, G), lambda i: (0, 0)),
      out_shape=jax.ShapeDtypeStruct((1, G), jnp.float32),
  )(gi3)


def _head_body(sums_ref, cnt_ref, w_ref, b_ref, out_ref):
  s = sums_ref[0:G, :] + sums_ref[G:2 * G, :]
  c = jnp.reshape(cnt_ref[0, :], (G, 1))
  pooled = s / jnp.maximum(c, 1.0)
  out_ref[...] = (
      jnp.dot(pooled, w_ref[...], preferred_element_type=jnp.float32)
      + b_ref[...])


@jax.jit
def _head(sums, cnts, W, b):
  return pl.pallas_call(
      _head_body,
      out_shape=jax.ShapeDtypeStruct((G, TC_OUT), jnp.float32),
  )(sums, cnts, W, b.reshape(1, TC_OUT))


def kernel(node_representation, graph_index, W, b):
  gi = graph_index.astype(jnp.int32)
  gi_r = gi.reshape(NB, NCH, CH)
  gi3 = gi.reshape(NHB, 1, HB)
  sums = _segment_accumulate(node_representation, gi_r)
  cnts = _hist(gi3)
  out = _head(sums, cnts, W, b)
  return out.reshape(-1, TC_OUT // 2, 2)


# speedup vs baseline: 8.2330x; 1.0052x over previous
import functools

import jax
import jax.numpy as jnp
from jax import lax
from jax.experimental import pallas as pl
from jax.experimental.pallas import tpu as pltpu
from jax.experimental.pallas import tpu_sc as plsc

N = 100000
D = 128
G = 512
TC_OUT = 24

NC = 2
NS = 16
NW = NC * NS

B = 400
CH = 100
NCH = B // CH
NB = N // B
ROWS_PER_TILE = G // NS

HB = 2000
NHB = N // HB


def _sc_body(nodes_hbm, gi_hbm, sums_hbm,
             rows_v0, rows_v1, idx_v0, idx_v1, zrow_v, acc_sh,
             lsem0, lsem1, ssem0, ssem1):
  cid = lax.axis_index("c")
  sid = lax.axis_index("s")
  wid = cid * NS + sid

  def fill_zrow(i, _):
    for j in range(D // 16):
      zrow_v[i, pl.ds(j * 16, 16)] = jnp.zeros((16,), jnp.float32)
    return 0
  lax.fori_loop(0, ROWS_PER_TILE, fill_zrow, 0)

  pltpu.sync_copy(zrow_v, acc_sh.at[pl.ds(sid * ROWS_PER_TILE, ROWS_PER_TILE)])
  plsc.subcore_barrier()

  nbase = NB // NW
  nrem = NB - nbase * NW
  start = nbase * wid + jnp.minimum(wid, nrem)
  nb = nbase + jnp.where(wid < nrem, 1, 0)

  def issue_load(k, rows_v, idx_v, lsem):
    bt = start + k
    pltpu.async_copy(gi_hbm.at[bt], idx_v, lsem)
    pltpu.async_copy(nodes_hbm.at[pl.ds(bt * B, B)], rows_v, lsem)

  def half(k, rows_v, idx_v, lsem, ssem):
    pltpu.make_async_copy(gi_hbm.at[0], idx_v, lsem).wait()
    pltpu.make_async_copy(nodes_hbm.at[pl.ds(0, B)], rows_v, lsem).wait()
    descs = []
    for j in range(NCH):
      descs.append(pltpu.async_copy(rows_v.at[pl.ds(j * CH, CH)],
                                    acc_sh.at[idx_v.at[j]], ssem, add=True))
    for d_ in descs:
      d_.wait()

    @pl.when(k + 2 < nb)
    def _():
      issue_load(k + 2, rows_v, idx_v, lsem)

  issue_load(0, rows_v0, idx_v0, lsem0)
  issue_load(1, rows_v1, idx_v1, lsem1)

  def pair_body(k2, _):
    k = 2 * k2

    @pl.when(k < nb)
    def _():
      half(k, rows_v0, idx_v0, lsem0, ssem0)

    @pl.when(k + 1 < nb)
    def _():
      half(k + 1, rows_v1, idx_v1, lsem1, ssem1)
    return 0

  lax.fori_loop(0, (nb + 1) // 2, pair_body, 0)
  plsc.subcore_barrier()

  out_base = cid * G + sid * ROWS_PER_TILE
  pltpu.sync_copy(acc_sh.at[pl.ds(sid * ROWS_PER_TILE, ROWS_PER_TILE)],
                  sums_hbm.at[pl.ds(out_base, ROWS_PER_TILE)])


@jax.jit
def _segment_accumulate(nodes, gi_r):
  mesh = plsc.VectorSubcoreMesh(core_axis_name="c", subcore_axis_name="s",
                                num_cores=NC, num_subcores=NS)
  return pl.kernel(
      _sc_body,
      out_type=jax.ShapeDtypeStruct((NC * G, D), jnp.float32),
      mesh=mesh,
      scratch_types=[
          pltpu.VMEM((B, D), jnp.float32),
          pltpu.VMEM((B, D), jnp.float32),
          pltpu.VMEM((NCH, CH), jnp.int32),
          pltpu.VMEM((NCH, CH), jnp.int32),
          pltpu.VMEM((ROWS_PER_TILE, D), jnp.float32),
          pltpu.VMEM_SHARED((G, D), jnp.float32),
          pltpu.SemaphoreType.DMA,
          pltpu.SemaphoreType.DMA,
          pltpu.SemaphoreType.DMA,
          pltpu.SemaphoreType.DMA,
      ],
  )(nodes, gi_r)


def _hist_body(gi_ref, cnt_ref):
  i = pl.program_id(0)
  gi_blk = gi_ref[0, 0, :]
  eq = (gi_blk[:, None] == lax.broadcasted_iota(jnp.int32, (HB, G), 1))
  bc = jnp.sum(eq.astype(jnp.float32), axis=0)

  @pl.when(i == 0)
  def _():
    cnt_ref[0, :] = bc

  @pl.when(i > 0)
  def _():
    cnt_ref[0, :] = cnt_ref[0, :] + bc


@jax.jit
def _hist(gi3):
  return pl.pallas_call(
      _hist_body,
      grid=(NHB,),
      in_specs=[pl.BlockSpec((1, 1, HB), lambda i: (i, 0, 0))],
      out_specs=pl.BlockSpec((1, G), lambda i: (0, 0)),
      out_shape=jax.ShapeDtypeStruct((1, G), jnp.float32),
  )(gi3)


def _head_body(sums_ref, cnt_ref, w_ref, b_ref, out_ref):
  s = sums_ref[0:G, :] + sums_ref[G:2 * G, :]
  c = jnp.reshape(cnt_ref[0, :], (G, 1))
  pooled = s / jnp.maximum(c, 1.0)
  out_ref[...] = (
      jnp.dot(pooled, w_ref[...], preferred_element_type=jnp.float32)
      + b_ref[...])


@jax.jit
def _head(sums, cnts, W, b):
  return pl.pallas_call(
      _head_body,
      out_shape=jax.ShapeDtypeStruct((G, TC_OUT), jnp.float32),
  )(sums, cnts, W, b.reshape(1, TC_OUT))


def kernel(node_representation, graph_index, W, b):
  gi = graph_index.astype(jnp.int32)
  gi_r = gi.reshape(NB, NCH, CH)
  gi3 = gi.reshape(NHB, 1, HB)
  sums = _segment_accumulate(node_representation, gi_r)
  cnts = _hist(gi3)
  out = _head(sums, cnts, W, b)
  return out.reshape(-1, TC_OUT // 2, 2)
